# baseline (speedup 1.0000x reference)
import jax
import jax.numpy as jnp
from jax.experimental import pallas as pl


_NUM_CODEBOOKS = 4
_CODEBOOK_SIZE = 1024
_CODEBOOK_DIM = 64
_COMMITMENT_COST = 0.25
_TILE = 4096
_NCHAINS = 8


def _rvq_kernel(x_ref, cb_ref, cbt2_ref, iota_ref, q_ref, idx_ref, loss_ref):
    cb = cb_ref[...]
    cbt2 = cbt2_ref[...]
    c2 = 0.25 * jnp.sum(cbt2 * cbt2, axis=0, keepdims=True)
    chunk = _TILE // _NCHAINS
    iota = iota_ref[...]

    @pl.when(pl.program_id(0) == 0)
    def _():
        loss_ref[...] = jnp.zeros_like(loss_ref)

    xs = [x_ref[c * chunk:(c + 1) * chunk, :] for c in range(_NCHAINS)]
    rs = list(xs)
    losses = []
    idx_rows = [[] for _ in range(_NCHAINS)]
    for _ in range(_NUM_CODEBOOKS):
        for c in range(_NCHAINS):
            r = rs[c]
            r2 = jnp.sum(r * r, axis=1, keepdims=True)
            scores = (
                r2 + jnp.dot(r, cbt2, preferred_element_type=jnp.float32)
            ) + c2
            smin = jnp.min(scores, axis=1, keepdims=True)
            idx = jnp.min(
                jnp.where(scores == smin, iota, float(_CODEBOOK_SIZE)),
                axis=1,
                keepdims=True,
            )
            onehot = (iota == idx).astype(jnp.float32)
            q = jnp.dot(onehot, cb, preferred_element_type=jnp.float32)
            losses.append(jnp.sum(smin))
            rs[c] = r - q
            idx_rows[c].append(idx[:, 0])

    for c in range(_NCHAINS):
        q_ref[c * chunk:(c + 1) * chunk, :] = xs[c] - rs[c]
        idx_ref[:, c * chunk:(c + 1) * chunk] = jnp.stack(
            idx_rows[c], axis=0
        ).astype(jnp.int32)
    loss_ref[...] = loss_ref[...] + sum(losses)


@jax.jit
def kernel(inputs, codebook):
    orig_shape = inputs.shape
    x = inputs.reshape(-1, _CODEBOOK_DIM)
    n = x.shape[0]
    grid = n // _TILE
    cbt2 = -2.0 * codebook.T
    chunk = _TILE // _NCHAINS
    iota_f = jnp.broadcast_to(
        jnp.arange(_CODEBOOK_SIZE, dtype=jnp.float32)[None, :],
        (chunk, _CODEBOOK_SIZE),
    )
    q, idx, loss = pl.pallas_call(
        _rvq_kernel,
        grid=(grid,),
        in_specs=[
            pl.BlockSpec((_TILE, _CODEBOOK_DIM), lambda i: (i, 0)),
            pl.BlockSpec((_CODEBOOK_SIZE, _CODEBOOK_DIM), lambda i: (0, 0)),
            pl.BlockSpec((_CODEBOOK_DIM, _CODEBOOK_SIZE), lambda i: (0, 0)),
            pl.BlockSpec(
                (_TILE // _NCHAINS, _CODEBOOK_SIZE), lambda i: (0, 0)
            ),
        ],
        out_specs=[
            pl.BlockSpec((_TILE, _CODEBOOK_DIM), lambda i: (i, 0)),
            pl.BlockSpec((_NUM_CODEBOOKS, _TILE), lambda i: (0, i)),
            pl.BlockSpec((1, 1), lambda i: (0, 0)),
        ],
        out_shape=[
            jax.ShapeDtypeStruct((n, _CODEBOOK_DIM), jnp.float32),
            jax.ShapeDtypeStruct((_NUM_CODEBOOKS, n), jnp.int32),
            jax.ShapeDtypeStruct((1, 1), jnp.float32),
        ],
    )(x, codebook, cbt2, iota_f)
    quantized_out = q.reshape(orig_shape)
    all_indices = idx.T
    vq_loss = (
        (1.0 + _COMMITMENT_COST) * loss[0, 0] / jnp.float32(n * _CODEBOOK_DIM)
    )
    return quantized_out, all_indices, vq_loss


# speedup vs baseline: 1.0497x; 1.0497x over previous
import jax
import jax.numpy as jnp
from jax.experimental import pallas as pl
from jax.experimental.pallas import tpu as pltpu


_NUM_CODEBOOKS = 4
_CODEBOOK_SIZE = 1024
_CODEBOOK_DIM = 64
_COMMITMENT_COST = 0.25
_TILE = 4096
_NCHAINS = 8


def _rvq_kernel(x_ref, cb_ref, cbt2_ref, q_ref, idx_ref, loss_ref):
    cb = cb_ref[...]
    cbt2 = cbt2_ref[...]
    c2 = 0.25 * jnp.sum(cbt2 * cbt2, axis=0, keepdims=True)
    chunk = _TILE // _NCHAINS
    iota = jax.lax.broadcasted_iota(
        jnp.int32, (chunk, _CODEBOOK_SIZE), 1
    ).astype(jnp.float32)

    xs = [x_ref[c * chunk:(c + 1) * chunk, :] for c in range(_NCHAINS)]
    rs = list(xs)
    losses = []
    idx_rows = [[] for _ in range(_NCHAINS)]
    for _ in range(_NUM_CODEBOOKS):
        for c in range(_NCHAINS):
            r = rs[c]
            r2 = jnp.sum(r * r, axis=1, keepdims=True)
            scores = (
                r2 + jnp.dot(r, cbt2, preferred_element_type=jnp.float32)
            ) + c2
            smin = jnp.min(scores, axis=1, keepdims=True)
            idx = jnp.min(
                jnp.where(scores == smin, iota, float(_CODEBOOK_SIZE)),
                axis=1,
                keepdims=True,
            )
            onehot = (iota == idx).astype(jnp.float32)
            q = jnp.dot(onehot, cb, preferred_element_type=jnp.float32)
            losses.append(jnp.sum(smin))
            rs[c] = r - q
            idx_rows[c].append(idx[:, 0])

    for c in range(_NCHAINS):
        q_ref[c * chunk:(c + 1) * chunk, :] = xs[c] - rs[c]
        idx_ref[:, c * chunk:(c + 1) * chunk] = jnp.stack(
            idx_rows[c], axis=0
        ).astype(jnp.int32)
    loss_ref[...] = sum(losses) + jnp.zeros((1, 1, 1), jnp.float32)


@jax.jit
def kernel(inputs, codebook):
    orig_shape = inputs.shape
    x = inputs.reshape(-1, _CODEBOOK_DIM)
    n = x.shape[0]
    grid = n // _TILE
    cbt2 = -2.0 * codebook.T
    q, idx, loss = pl.pallas_call(
        _rvq_kernel,
        grid=(grid,),
        in_specs=[
            pl.BlockSpec((_TILE, _CODEBOOK_DIM), lambda i: (i, 0)),
            pl.BlockSpec((_CODEBOOK_SIZE, _CODEBOOK_DIM), lambda i: (0, 0)),
            pl.BlockSpec((_CODEBOOK_DIM, _CODEBOOK_SIZE), lambda i: (0, 0)),
        ],
        out_specs=[
            pl.BlockSpec((_TILE, _CODEBOOK_DIM), lambda i: (i, 0)),
            pl.BlockSpec((_NUM_CODEBOOKS, _TILE), lambda i: (0, i)),
            pl.BlockSpec((1, 1, 1), lambda i: (i, 0, 0)),
        ],
        out_shape=[
            jax.ShapeDtypeStruct((n, _CODEBOOK_DIM), jnp.float32),
            jax.ShapeDtypeStruct((_NUM_CODEBOOKS, n), jnp.int32),
            jax.ShapeDtypeStruct((grid, 1, 1), jnp.float32),
        ],
        compiler_params=pltpu.CompilerParams(
            dimension_semantics=("parallel",),
        ),
    )(x, codebook, cbt2)
    quantized_out = q.reshape(orig_shape)
    all_indices = idx.T
    vq_loss = (
        (1.0 + _COMMITMENT_COST)
        * jnp.sum(loss)
        / jnp.float32(n * _CODEBOOK_DIM)
    )
    return quantized_out, all_indices, vq_loss
